# baseline (speedup 1.0000x reference)
import functools

import jax
import jax.numpy as jnp
from jax.experimental import pallas as pl
from jax.experimental.pallas import tpu as pltpu

_BN = 512


def _cd_body(x1_ref, x2t_ref, d1_ref, d2_ref):
    nb = pl.program_id(1)
    x1 = x1_ref[0]
    x2t = x2t_ref[0]
    d = jax.lax.dot_general(
        x1, x2t, dimension_numbers=(((1,), (0,)), ((), ())),
        preferred_element_type=jnp.float32,
        precision=jax.lax.Precision.HIGHEST)
    d1_ref[0] = jnp.min(d, axis=1, keepdims=True)
    part = jnp.min(d, axis=0, keepdims=True)

    @pl.when(nb == 0)
    def _():
        d2_ref[0] = part

    @pl.when(nb > 0)
    def _():
        d2_ref[0] = jnp.minimum(d2_ref[0], part)


@jax.jit
def kernel(xyz1, xyz2):
    B, N, _ = xyz1.shape
    M = xyz2.shape[1]
    x1s = jnp.sum(xyz1 * xyz1, axis=-1, keepdims=True)
    x2s = jnp.sum(xyz2 * xyz2, axis=-1, keepdims=True)
    ones1 = jnp.ones((B, N, 1), jnp.float32)
    ones2 = jnp.ones((B, M, 1), jnp.float32)
    zeros1 = jnp.zeros((B, N, 3), jnp.float32)
    zeros2 = jnp.zeros((B, M, 3), jnp.float32)
    x1a = jnp.concatenate(
        [-2.0 * xyz1, x1s, ones1, zeros1], axis=-1)
    x2a = jnp.transpose(
        jnp.concatenate([xyz2, ones2, x2s, zeros2], axis=-1),
        (0, 2, 1))
    grid = (B, N // _BN)
    d1, d2 = pl.pallas_call(
        _cd_body,
        grid=grid,
        in_specs=[
            pl.BlockSpec((1, _BN, 8), lambda b, i: (b, i, 0)),
            pl.BlockSpec((1, 8, M), lambda b, i: (b, 0, 0)),
        ],
        out_specs=[
            pl.BlockSpec((1, _BN, 1), lambda b, i: (b, i, 0)),
            pl.BlockSpec((1, 1, M), lambda b, i: (b, 0, 0)),
        ],
        out_shape=[
            jax.ShapeDtypeStruct((B, N, 1), jnp.float32),
            jax.ShapeDtypeStruct((B, 1, M), jnp.float32),
        ],
        compiler_params=pltpu.CompilerParams(
            dimension_semantics=("parallel", "arbitrary")),
    )(x1a, x2a)
    return d1.reshape(B, N), d2.reshape(B, M)


# speedup vs baseline: 3.5636x; 3.5636x over previous
import functools

import jax
import jax.numpy as jnp
from jax.experimental import pallas as pl
from jax.experimental.pallas import tpu as pltpu

_BN = 512


def _cd_body(x1_ref, x2m_ref, d1_ref, d2_ref):
    nb = pl.program_id(1)
    x1 = x1_ref[0]
    x2m = x2m_ref[0]
    x1s = jnp.sum(x1 * x1, axis=1, keepdims=True)
    x2s = 0.25 * jnp.sum(x2m * x2m, axis=0, keepdims=True)
    inner2 = jax.lax.dot_general(
        x1, x2m, dimension_numbers=(((1,), (0,)), ((), ())),
        preferred_element_type=jnp.float32)
    d1_ref[0] = jnp.min(inner2 + x2s, axis=1, keepdims=True) + x1s
    part = jnp.min(inner2 + x1s, axis=0, keepdims=True) + x2s

    @pl.when(nb == 0)
    def _():
        d2_ref[0] = part

    @pl.when(nb > 0)
    def _():
        d2_ref[0] = jnp.minimum(d2_ref[0], part)


@jax.jit
def kernel(xyz1, xyz2):
    B, N, _ = xyz1.shape
    M = xyz2.shape[1]
    x2m = jnp.transpose(-2.0 * xyz2, (0, 2, 1))
    grid = (B, N // _BN)
    d1, d2 = pl.pallas_call(
        _cd_body,
        grid=grid,
        in_specs=[
            pl.BlockSpec((1, _BN, 3), lambda b, i: (b, i, 0)),
            pl.BlockSpec((1, 3, M), lambda b, i: (b, 0, 0)),
        ],
        out_specs=[
            pl.BlockSpec((1, _BN, 1), lambda b, i: (b, i, 0)),
            pl.BlockSpec((1, 1, M), lambda b, i: (b, 0, 0)),
        ],
        out_shape=[
            jax.ShapeDtypeStruct((B, N, 1), jnp.float32),
            jax.ShapeDtypeStruct((B, 1, M), jnp.float32),
        ],
        compiler_params=pltpu.CompilerParams(
            dimension_semantics=("parallel", "arbitrary")),
    )(xyz1, x2m)
    return d1.reshape(B, N), d2.reshape(B, M)


# speedup vs baseline: 3.7498x; 1.0522x over previous
import functools

import jax
import jax.numpy as jnp
from jax.experimental import pallas as pl
from jax.experimental.pallas import tpu as pltpu

_BN = 1024


def _cd_body(x1_ref, x2m_ref, d1_ref, d2_ref):
    nb = pl.program_id(1)
    x1 = x1_ref[0]
    x2m = x2m_ref[0]
    x1s = jnp.sum(x1 * x1, axis=1, keepdims=True)
    x2s = 0.25 * jnp.sum(x2m * x2m, axis=0, keepdims=True)
    inner2 = jax.lax.dot_general(
        x1, x2m, dimension_numbers=(((1,), (0,)), ((), ())),
        preferred_element_type=jnp.float32)
    d1_ref[0] = jnp.min(inner2 + x2s, axis=1, keepdims=True) + x1s
    part = jnp.min(inner2 + x1s, axis=0, keepdims=True) + x2s

    @pl.when(nb == 0)
    def _():
        d2_ref[0] = part

    @pl.when(nb > 0)
    def _():
        d2_ref[0] = jnp.minimum(d2_ref[0], part)


@jax.jit
def kernel(xyz1, xyz2):
    B, N, _ = xyz1.shape
    M = xyz2.shape[1]
    x2m = jnp.transpose(-2.0 * xyz2, (0, 2, 1))
    grid = (B, N // _BN)
    d1, d2 = pl.pallas_call(
        _cd_body,
        grid=grid,
        in_specs=[
            pl.BlockSpec((1, _BN, 3), lambda b, i: (b, i, 0)),
            pl.BlockSpec((1, 3, M), lambda b, i: (b, 0, 0)),
        ],
        out_specs=[
            pl.BlockSpec((1, _BN, 1), lambda b, i: (b, i, 0)),
            pl.BlockSpec((1, 1, M), lambda b, i: (b, 0, 0)),
        ],
        out_shape=[
            jax.ShapeDtypeStruct((B, N, 1), jnp.float32),
            jax.ShapeDtypeStruct((B, 1, M), jnp.float32),
        ],
        compiler_params=pltpu.CompilerParams(
            dimension_semantics=("parallel", "arbitrary")),
    )(xyz1, x2m)
    return d1.reshape(B, N), d2.reshape(B, M)


# speedup vs baseline: 3.9095x; 1.0426x over previous
import functools

import jax
import jax.numpy as jnp
from jax.experimental import pallas as pl
from jax.experimental.pallas import tpu as pltpu

_BN = 2048


def _cd_body(x1_ref, x2m_ref, d1_ref, d2_ref):
    nb = pl.program_id(1)
    x1 = x1_ref[0]
    x2m = x2m_ref[0]
    x1s = jnp.sum(x1 * x1, axis=1, keepdims=True)
    x2s = 0.25 * jnp.sum(x2m * x2m, axis=0, keepdims=True)
    inner2 = jax.lax.dot_general(
        x1, x2m, dimension_numbers=(((1,), (0,)), ((), ())),
        preferred_element_type=jnp.float32)
    d1_ref[0] = jnp.min(inner2 + x2s, axis=1, keepdims=True) + x1s
    part = jnp.min(inner2 + x1s, axis=0, keepdims=True) + x2s

    @pl.when(nb == 0)
    def _():
        d2_ref[0] = part

    @pl.when(nb > 0)
    def _():
        d2_ref[0] = jnp.minimum(d2_ref[0], part)


@jax.jit
def kernel(xyz1, xyz2):
    B, N, _ = xyz1.shape
    M = xyz2.shape[1]
    x2m = jnp.transpose(-2.0 * xyz2, (0, 2, 1))
    grid = (B, N // _BN)
    d1, d2 = pl.pallas_call(
        _cd_body,
        grid=grid,
        in_specs=[
            pl.BlockSpec((1, _BN, 3), lambda b, i: (b, i, 0)),
            pl.BlockSpec((1, 3, M), lambda b, i: (b, 0, 0)),
        ],
        out_specs=[
            pl.BlockSpec((1, _BN, 1), lambda b, i: (b, i, 0)),
            pl.BlockSpec((1, 1, M), lambda b, i: (b, 0, 0)),
        ],
        out_shape=[
            jax.ShapeDtypeStruct((B, N, 1), jnp.float32),
            jax.ShapeDtypeStruct((B, 1, M), jnp.float32),
        ],
        compiler_params=pltpu.CompilerParams(
            dimension_semantics=("parallel", "arbitrary")),
    )(xyz1, x2m)
    return d1.reshape(B, N), d2.reshape(B, M)
